# baseline (speedup 1.0000x reference)
import functools

import jax
import jax.numpy as jnp
from jax import lax
from jax.experimental import pallas as pl
from jax.experimental.pallas import tpu as pltpu
from jax.experimental.pallas import tpu_sc as plsc

NC, NS, L = 2, 16, 16
NW = NC * NS
CHUNK = 80
BM = 1024
ZR = 128


def _mesh():
    return plsc.VectorSubcoreMesh(core_axis_name="c", subcore_axis_name="s",
                                  num_cores=NC, num_subcores=NS)


def _deg_call(dst4, n):
    nw, nb, w, chunk = dst4.shape
    rpt = n // NS
    zr = ZR
    w16 = 16

    @functools.partial(
        pl.kernel,
        out_type=jax.ShapeDtypeStruct((NC, n, w16), jnp.float32),
        mesh=_mesh(),
        scratch_types=[
            pltpu.VMEM((nb, w, chunk), jnp.int32),
            pltpu.VMEM((chunk, w16), jnp.float32),
            pltpu.VMEM((zr, w16), jnp.float32),
            pltpu.VMEM_SHARED((n, w16), jnp.float32),
        ],
    )
    def deg_k(dst_hbm, out_hbm, d_idx, ones_v, zbuf, acc):
        cid = lax.axis_index("c")
        sid = lax.axis_index("s")
        wid = sid * NC + cid
        ones = jnp.ones((L,), jnp.float32)
        zeros = jnp.zeros((L,), jnp.float32)

        def fill_ones(i, _):
            ones_v[i, pl.ds(0, L)] = ones
            return 0

        lax.fori_loop(0, chunk, fill_ones, 0)

        def fill_z(i, _):
            zbuf[i, pl.ds(0, L)] = zeros
            return 0

        lax.fori_loop(0, zr, fill_z, 0)
        for r in range(rpt // zr):
            pltpu.sync_copy(zbuf, acc.at[pl.ds(sid * rpt + r * zr, zr)])
        pltpu.sync_copy(dst_hbm.at[wid], d_idx)
        plsc.subcore_barrier()

        def body_b(b, _):
            def body_c(c, _):
                pltpu.sync_copy(ones_v, acc.at[d_idx.at[b, c]], add=True)
                return 0
            return lax.fori_loop(0, w, body_c, 0)

        lax.fori_loop(0, nb, body_b, 0)
        plsc.subcore_barrier()
        pltpu.sync_copy(acc.at[pl.ds(sid * rpt, rpt)],
                        out_hbm.at[cid, pl.ds(sid * rpt, rpt)])

    return deg_k(dst4)


def _seg_call(g, src4, dst4):
    n, d = g.shape
    nw, nb, w, chunk = src4.shape
    rpt = n // NS
    zr = ZR

    @functools.partial(
        pl.kernel,
        out_type=jax.ShapeDtypeStruct((NC, n, d), jnp.float32),
        mesh=_mesh(),
        scratch_types=[
            pltpu.VMEM((w, chunk), jnp.int32),
            pltpu.VMEM((w, chunk), jnp.int32),
            pltpu.VMEM((chunk, d), jnp.float32),
            pltpu.VMEM((chunk, d), jnp.float32),
            pltpu.VMEM((ZR // NS, d), jnp.float32),
            pltpu.VMEM_SHARED((zr, d), jnp.float32),
            pltpu.VMEM_SHARED((n, d), jnp.float32),
            pltpu.SemaphoreType.DMA,
            pltpu.SemaphoreType.DMA,
            pltpu.SemaphoreType.DMA,
            pltpu.SemaphoreType.DMA,
        ],
    )
    def seg_k(g_hbm, src_hbm, dst_hbm, out_hbm,
              s_idx, d_idx, buf0, buf1, zloc, zbuf, acc,
              gsem0, gsem1, ssem0, ssem1):
        cid = lax.axis_index("c")
        sid = lax.axis_index("s")
        wid = sid * NC + cid
        zeros = jnp.zeros((L,), jnp.float32)
        zps = zr // NS

        def zrow(i, _):
            def zcol(j, _):
                zloc[i, pl.ds(j * L, L)] = zeros
                return 0
            return lax.fori_loop(0, d // L, zcol, 0)

        lax.fori_loop(0, zps, zrow, 0)
        pltpu.sync_copy(zloc, zbuf.at[pl.ds(sid * zps, zps)])
        plsc.subcore_barrier()
        for r in range(rpt // zr):
            pltpu.sync_copy(zbuf, acc.at[pl.ds(sid * rpt + r * zr, zr)])
        plsc.subcore_barrier()


        def gwait(c, buf, gsem):
            pltpu.make_async_copy(g_hbm.at[s_idx.at[c]], buf, gsem).wait()

        def swait(c, buf, ssem):
            pltpu.make_async_copy(buf, acc.at[d_idx.at[c]], ssem).wait()

        for b in range(nb):
            pltpu.sync_copy(src_hbm.at[wid, b], s_idx)
            pltpu.sync_copy(dst_hbm.at[wid, b], d_idx)
            pltpu.async_copy(g_hbm.at[s_idx.at[0]], buf0, gsem0)
            gwait(0, buf0, gsem0)
            pltpu.async_copy(buf0, acc.at[d_idx.at[0]], ssem0, add=True)
            pltpu.async_copy(g_hbm.at[s_idx.at[1]], buf1, gsem1)

            def body(k, _):
                c0 = 2 * k + 1
                gwait(c0, buf1, gsem1)
                pltpu.async_copy(buf1, acc.at[d_idx.at[c0]], ssem1, add=True)
                swait(c0 - 1, buf0, ssem0)
                pltpu.async_copy(g_hbm.at[s_idx.at[c0 + 1]], buf0, gsem0)
                gwait(c0 + 1, buf0, gsem0)
                pltpu.async_copy(buf0, acc.at[d_idx.at[c0 + 1]], ssem0,
                                 add=True)
                swait(c0, buf1, ssem1)
                cn = jnp.minimum(c0 + 2, w - 1)
                pltpu.async_copy(g_hbm.at[s_idx.at[cn]], buf1, gsem1)
                return 0

            lax.fori_loop(0, (w - 1) // 2, body, 0)
            gwait(w - 1, buf1, gsem1)
            swait(w - 1, buf0, ssem0)

        plsc.subcore_barrier()
        pltpu.sync_copy(acc.at[pl.ds(sid * rpt, rpt)],
                        out_hbm.at[cid, pl.ds(sid * rpt, rpt)])

    return seg_k(g, src4, dst4)


def _dinv_of(dp_ref):
    deg = dp_ref[0, :, 0:1] + dp_ref[1, :, 0:1] + 1.0
    return lax.rsqrt(deg)


def _tc_scale_matmul(x, w, degp):
    n, din = x.shape
    dh = w.shape[1]

    def body(x_ref, w_ref, dp_ref, g_ref):
        h = jnp.dot(x_ref[...], w_ref[...], preferred_element_type=jnp.float32)
        g_ref[...] = h * _dinv_of(dp_ref)

    return pl.pallas_call(
        body,
        grid=(n // BM,),
        in_specs=[
            pl.BlockSpec((BM, din), lambda i: (i, 0)),
            pl.BlockSpec((din, dh), lambda i: (0, 0)),
            pl.BlockSpec((NC, BM, 16), lambda i: (0, i, 0)),
        ],
        out_specs=pl.BlockSpec((BM, dh), lambda i: (i, 0)),
        out_shape=jax.ShapeDtypeStruct((n, dh), jnp.float32),
    )(x, w, degp)


def _tc_mid(agg1p, g1, degp, b1):
    n, dh = g1.shape

    def body(ap_ref, g1_ref, dp_ref, b1_ref, emb_ref, ge_ref):
        dinv = _dinv_of(dp_ref)
        agg = ap_ref[0] + ap_ref[1] + g1_ref[...]
        emb = jnp.maximum(agg * dinv + b1_ref[...][None, :], 0.0)
        emb_ref[...] = emb
        ge_ref[...] = emb * dinv

    return pl.pallas_call(
        body,
        grid=(n // BM,),
        in_specs=[
            pl.BlockSpec((NC, BM, dh), lambda i: (0, i, 0)),
            pl.BlockSpec((BM, dh), lambda i: (i, 0)),
            pl.BlockSpec((NC, BM, 16), lambda i: (0, i, 0)),
            pl.BlockSpec((dh,), lambda i: (0,)),
        ],
        out_specs=[
            pl.BlockSpec((BM, dh), lambda i: (i, 0)),
            pl.BlockSpec((BM, dh), lambda i: (i, 0)),
        ],
        out_shape=[
            jax.ShapeDtypeStruct((n, dh), jnp.float32),
            jax.ShapeDtypeStruct((n, dh), jnp.float32),
        ],
    )(agg1p, g1, degp, b1)


def _tc_final(agg2p, ge, degp, b2, w2):
    n, dh = ge.shape
    dout = w2.shape[1]

    def body(ap_ref, ge_ref, dp_ref, b2_ref, w2_ref, o_ref):
        dinv = _dinv_of(dp_ref)
        t = ap_ref[0] + ap_ref[1] + ge_ref[...]
        o = jnp.dot(t, w2_ref[...], preferred_element_type=jnp.float32)
        o = o * dinv + b2_ref[...][None, :]
        m = jnp.max(o, axis=1, keepdims=True)
        lse = jnp.log(jnp.sum(jnp.exp(o - m), axis=1, keepdims=True)) + m
        o_ref[...] = o - lse

    return pl.pallas_call(
        body,
        grid=(n // BM,),
        in_specs=[
            pl.BlockSpec((NC, BM, dh), lambda i: (0, i, 0)),
            pl.BlockSpec((BM, dh), lambda i: (i, 0)),
            pl.BlockSpec((NC, BM, 16), lambda i: (0, i, 0)),
            pl.BlockSpec((dout,), lambda i: (0,)),
            pl.BlockSpec((dh, dout), lambda i: (0, 0)),
        ],
        out_specs=pl.BlockSpec((BM, dout), lambda i: (i, 0)),
        out_shape=jax.ShapeDtypeStruct((n, dout), jnp.float32),
    )(agg2p, ge, degp, b2, w2)


def kernel(x, edge_index, W1, b1, W2, b2):
    n = x.shape[0]
    e = edge_index.shape[1]
    npad = -(-n // (NS * ZR)) * (NS * ZR)
    xp = jnp.pad(x, ((0, npad - n), (0, 0)))
    epw = e // NW
    nch = epw // CHUNK
    nb = 5
    w = nch // nb
    src4 = edge_index[0].reshape(NW, nb, w, CHUNK)
    dst4 = edge_index[1].reshape(NW, nb, w, CHUNK)

    degp = _deg_call(dst4, npad)
    g1 = _tc_scale_matmul(xp, W1, degp)
    agg1p = _seg_call(g1, src4, dst4)
    emb, ge = _tc_mid(agg1p, g1, degp, b1)
    agg2p = _seg_call(ge, src4, dst4)
    out = _tc_final(agg2p, ge, degp, b2, W2)
    return out[:n], emb[:n]


# speedup vs baseline: 2.7631x; 2.7631x over previous
import functools

import jax
import jax.numpy as jnp
from jax import lax
from jax.experimental import pallas as pl
from jax.experimental.pallas import tpu as pltpu
from jax.experimental.pallas import tpu_sc as plsc

NC, NS, L = 2, 16, 16
NW = NC * NS
CHUNK = 80
BM = 1024
ZR = 128


def _mesh():
    return plsc.VectorSubcoreMesh(core_axis_name="c", subcore_axis_name="s",
                                  num_cores=NC, num_subcores=NS)


def _deg_call(dst4, n):
    nw, nb, w, chunk = dst4.shape
    rpt = n // NS
    zr = ZR
    w16 = 16

    @functools.partial(
        pl.kernel,
        out_type=jax.ShapeDtypeStruct((NC, n, w16), jnp.float32),
        mesh=_mesh(),
        scratch_types=[
            pltpu.VMEM((nb, w, chunk), jnp.int32),
            pltpu.VMEM((chunk, w16), jnp.float32),
            pltpu.VMEM((zr, w16), jnp.float32),
            pltpu.VMEM_SHARED((n, w16), jnp.float32),
        ],
    )
    def deg_k(dst_hbm, out_hbm, d_idx, ones_v, zbuf, acc):
        cid = lax.axis_index("c")
        sid = lax.axis_index("s")
        wid = sid * NC + cid
        ones = jnp.ones((L,), jnp.float32)
        zeros = jnp.zeros((L,), jnp.float32)

        def fill_ones(i, _):
            ones_v[i, pl.ds(0, L)] = ones
            return 0

        lax.fori_loop(0, chunk, fill_ones, 0)

        def fill_z(i, _):
            zbuf[i, pl.ds(0, L)] = zeros
            return 0

        lax.fori_loop(0, zr, fill_z, 0)
        for r in range(rpt // zr):
            pltpu.sync_copy(zbuf, acc.at[pl.ds(sid * rpt + r * zr, zr)])
        pltpu.sync_copy(dst_hbm.at[wid], d_idx)
        plsc.subcore_barrier()

        def body_b(b, _):
            def body_c(c, _):
                pltpu.sync_copy(ones_v, acc.at[d_idx.at[b, c]], add=True)
                return 0
            return lax.fori_loop(0, w, body_c, 0)

        lax.fori_loop(0, nb, body_b, 0)
        plsc.subcore_barrier()
        pltpu.sync_copy(acc.at[pl.ds(sid * rpt, rpt)],
                        out_hbm.at[cid, pl.ds(sid * rpt, rpt)])

    return deg_k(dst4)


def _seg_call(g, src4, dst4, zeros):
    n, d = g.shape
    nw, nb, w, chunk = src4.shape
    rpt = n // NS
    zr = ZR
    dt = g.dtype

    @functools.partial(
        pl.kernel,
        out_type=jax.ShapeDtypeStruct((NC, n, d), dt),
        mesh=_mesh(),
        scratch_types=[
            pltpu.VMEM((w, chunk), jnp.int32),
            pltpu.VMEM((w, chunk), jnp.int32),
            pltpu.VMEM((chunk, d), dt),
            pltpu.VMEM((chunk, d), dt),
            pltpu.VMEM_SHARED((n, d), dt),
            pltpu.SemaphoreType.DMA,
            pltpu.SemaphoreType.DMA,
        ],
    )
    def seg_k(g_hbm, src_hbm, dst_hbm, z_hbm, out_hbm,
              s_idx, d_idx, buf0, buf1, acc, gsem0, gsem1):
        cid = lax.axis_index("c")
        sid = lax.axis_index("s")
        wid = sid * NC + cid

        for r in range(rpt // zr):
            pltpu.sync_copy(z_hbm, acc.at[pl.ds(sid * rpt + r * zr, zr)])
        plsc.subcore_barrier()

        for b in range(nb):
            pltpu.sync_copy(src_hbm.at[wid, b], s_idx)
            pltpu.sync_copy(dst_hbm.at[wid, b], d_idx)
            pltpu.async_copy(g_hbm.at[s_idx.at[0]], buf0, gsem0)

            def body(k, _):
                c0 = 2 * k
                pltpu.async_copy(g_hbm.at[s_idx.at[c0 + 1]], buf1, gsem1)
                pltpu.make_async_copy(g_hbm.at[s_idx.at[c0]], buf0,
                                      gsem0).wait()
                pltpu.sync_copy(buf0, acc.at[d_idx.at[c0]], add=True)
                pltpu.async_copy(g_hbm.at[s_idx.at[c0 + 2]], buf0, gsem0)
                pltpu.make_async_copy(g_hbm.at[s_idx.at[c0 + 1]], buf1,
                                      gsem1).wait()
                pltpu.sync_copy(buf1, acc.at[d_idx.at[c0 + 1]], add=True)
                return 0

            lax.fori_loop(0, (w - 1) // 2, body, 0)
            pltpu.make_async_copy(g_hbm.at[s_idx.at[w - 1]], buf0,
                                  gsem0).wait()
            pltpu.sync_copy(buf0, acc.at[d_idx.at[w - 1]], add=True)

        plsc.subcore_barrier()
        pltpu.sync_copy(acc.at[pl.ds(sid * rpt, rpt)],
                        out_hbm.at[cid, pl.ds(sid * rpt, rpt)])

    return seg_k(g, src4, dst4, zeros)


def _dinv_of(dp_ref):
    deg = dp_ref[0, :, 0:1] + dp_ref[1, :, 0:1] + 1.0
    return lax.rsqrt(deg)


def _tc_scale_matmul(x, w, degp):
    n, din = x.shape
    dh = w.shape[1]

    def body(x_ref, w_ref, dp_ref, g_ref):
        h = jnp.dot(x_ref[...], w_ref[...], preferred_element_type=jnp.float32)
        g_ref[...] = h * _dinv_of(dp_ref)

    return pl.pallas_call(
        body,
        grid=(n // BM,),
        in_specs=[
            pl.BlockSpec((BM, din), lambda i: (i, 0)),
            pl.BlockSpec((din, dh), lambda i: (0, 0)),
            pl.BlockSpec((NC, BM, 16), lambda i: (0, i, 0)),
        ],
        out_specs=pl.BlockSpec((BM, dh), lambda i: (i, 0)),
        out_shape=jax.ShapeDtypeStruct((n, dh), jnp.float32),
    )(x, w, degp)


def _tc_mid(agg1p, g1, degp, b1):
    n, dh = g1.shape

    def body(ap_ref, g1_ref, dp_ref, b1_ref, emb_ref, ge_ref):
        dinv = _dinv_of(dp_ref)
        agg = ap_ref[0] + ap_ref[1] + g1_ref[...]
        emb = jnp.maximum(agg * dinv + b1_ref[...][None, :], 0.0)
        emb_ref[...] = emb
        ge_ref[...] = emb * dinv

    return pl.pallas_call(
        body,
        grid=(n // BM,),
        in_specs=[
            pl.BlockSpec((NC, BM, dh), lambda i: (0, i, 0)),
            pl.BlockSpec((BM, dh), lambda i: (i, 0)),
            pl.BlockSpec((NC, BM, 16), lambda i: (0, i, 0)),
            pl.BlockSpec((dh,), lambda i: (0,)),
        ],
        out_specs=[
            pl.BlockSpec((BM, dh), lambda i: (i, 0)),
            pl.BlockSpec((BM, dh), lambda i: (i, 0)),
        ],
        out_shape=[
            jax.ShapeDtypeStruct((n, dh), jnp.float32),
            jax.ShapeDtypeStruct((n, dh), jnp.float32),
        ],
    )(agg1p, g1, degp, b1)


def _tc_final(agg2p, emb, degp, b2, w2):
    n, dh = emb.shape
    dout = w2.shape[1]

    def body(ap_ref, emb_ref, dp_ref, b2_ref, w2_ref, o_ref):
        dinv = _dinv_of(dp_ref)
        t = ap_ref[0] + ap_ref[1] + emb_ref[...] * dinv
        o = jnp.dot(t, w2_ref[...], preferred_element_type=jnp.float32)
        o = o * dinv + b2_ref[...][None, :]
        m = jnp.max(o, axis=1, keepdims=True)
        lse = jnp.log(jnp.sum(jnp.exp(o - m), axis=1, keepdims=True)) + m
        o_ref[...] = o - lse

    return pl.pallas_call(
        body,
        grid=(n // BM,),
        in_specs=[
            pl.BlockSpec((NC, BM, dh), lambda i: (0, i, 0)),
            pl.BlockSpec((BM, dh), lambda i: (i, 0)),
            pl.BlockSpec((NC, BM, 16), lambda i: (0, i, 0)),
            pl.BlockSpec((dout,), lambda i: (0,)),
            pl.BlockSpec((dh, dout), lambda i: (0, 0)),
        ],
        out_specs=pl.BlockSpec((BM, dout), lambda i: (i, 0)),
        out_shape=jax.ShapeDtypeStruct((n, dout), jnp.float32),
    )(agg2p, emb, degp, b2, w2)


def kernel(x, edge_index, W1, b1, W2, b2):
    n = x.shape[0]
    e = edge_index.shape[1]
    npad = -(-n // (NS * ZR)) * (NS * ZR)
    xp = jnp.pad(x, ((0, npad - n), (0, 0)))
    epw = e // NW
    nch = epw // CHUNK
    nb = 5
    w = nch // nb
    src4 = edge_index[0].reshape(NW, nb, w, CHUNK)
    dst4 = edge_index[1].reshape(NW, nb, w, CHUNK)

    zeros = jnp.zeros((ZR, W1.shape[1]), jnp.float32)
    degp = _deg_call(dst4, npad)
    g1 = _tc_scale_matmul(xp, W1, degp)
    agg1p = _seg_call(g1, src4, dst4, zeros)
    emb, ge = _tc_mid(agg1p, g1, degp, b1)
    agg2p = _seg_call(ge, src4, dst4, zeros)
    out = _tc_final(agg2p, emb, degp, b2, W2)
    return out[:n], emb[:n]
